# baseline (speedup 1.0000x reference)
import functools

import jax
import jax.numpy as jnp
import numpy as np
from jax import lax
from jax.experimental import pallas as pl
from jax.experimental.pallas import tpu as pltpu
from jax.experimental.pallas import tpu_sc as plsc

_K = 16
_N = 8192
_B = 2
_TM = 512
_TM2 = 256
_NW = 32
_GCH = 128


def _window_list():
    out = []
    ws = 256
    cursor = 256
    while cursor < _N:
        ws = min(ws * 2, 1024)
        out.append((cursor, cursor, min(ws, _N - cursor)))
        cursor += ws
    return out


_WINDOWS = _window_list()
_M = sum(w[2] for w in _WINDOWS)


def _knn_body(t_ref, c_ref, o_ref, *, L, TM):
    b = pl.program_id(0)
    t = t_ref[0]
    t3 = t[:, 0:3]
    c3 = c_ref[0]
    tsq = jnp.sum(t3 * t3, axis=1, keepdims=True)
    ta = jnp.concatenate(
        [t3, tsq, jnp.ones((TM, 1), jnp.float32)], axis=1)
    csq = jnp.sum(c3 * c3, axis=0, keepdims=True)
    cb = jnp.concatenate(
        [-2.0 * c3, jnp.ones((1, L), jnp.float32), csq], axis=0)
    d = jnp.maximum(
        jnp.dot(ta, cb, preferred_element_type=jnp.float32), 0.0)
    iota = lax.broadcasted_iota(jnp.int32, (TM, L), 1)
    di = (lax.bitcast_convert_type(d, jnp.int32) & jnp.int32(~8191)) | iota
    L4 = L // 4
    a0 = di[:, 0:L4]
    a1 = di[:, L4:2 * L4]
    a2 = di[:, 2 * L4:3 * L4]
    a3 = di[:, 3 * L4:L]
    lo0 = jnp.minimum(a0, a1)
    hi0 = jnp.maximum(a0, a1)
    lo1 = jnp.minimum(a2, a3)
    hi1 = jnp.maximum(a2, a3)
    e = jnp.minimum(lo0, lo1)
    t1 = jnp.maximum(lo0, lo1)
    t2 = jnp.minimum(hi0, hi1)
    p3 = jnp.maximum(hi0, hi1)
    p1 = jnp.minimum(t1, t2)
    p2 = jnp.maximum(t1, t2)
    imax = jnp.int32(0x7FFFFFFF)
    cols = []
    for _ in range(_K):
        m = jnp.min(e, axis=1, keepdims=True)
        mask = e == m
        e = jnp.where(mask, p1, e)
        p1 = jnp.where(mask, p2, p1)
        p2 = jnp.where(mask, p3, p2)
        p3 = jnp.where(mask, imax, p3)
        cols.append(m & jnp.int32(8191))
    idx = jnp.concatenate(cols, axis=1)
    o_ref[0] = idx + b * _N


def _knn_window(targets, ctx_t, L):
    Mw = targets.shape[1]
    TM = min(_TM, Mw)
    grid = (_B, Mw // TM)
    return pl.pallas_call(
        functools.partial(_knn_body, L=L, TM=TM),
        grid=grid,
        in_specs=[
            pl.BlockSpec((1, TM, 4), lambda b, i: (b, i, 0)),
            pl.BlockSpec((1, 3, L), lambda b, i: (b, 0, 0)),
        ],
        out_specs=pl.BlockSpec((1, TM, _K), lambda b, i: (b, i, 0)),
        out_shape=jax.ShapeDtypeStruct((_B, Mw, _K), jnp.int32),
    )(targets, ctx_t)


def _sc_gather(table, idx3):
    nch = idx3.shape[1]
    total = _NW * nch * _GCH
    per_w = nch * _GCH
    mesh = plsc.VectorSubcoreMesh(core_axis_name="c", subcore_axis_name="s")

    @functools.partial(
        pl.kernel,
        mesh=mesh,
        compiler_params=pltpu.CompilerParams(use_tc_tiling_on_sc=False),
        out_type=jax.ShapeDtypeStruct((total, 16), jnp.float32),
        scratch_types=[
            pltpu.VMEM((nch, _GCH), jnp.int32),
        ] + [pltpu.VMEM((_GCH, 16), jnp.float32)] * 8
          + [pltpu.SemaphoreType.DMA] * 16,
    )
    def k(table_hbm, idx_hbm, out_hbm, idx_v, *bufs_and_sems):
        bufs = bufs_and_sems[0:8]
        gsem = bufs_and_sems[8:16]
        ssem = bufs_and_sems[16:24]
        cid = lax.axis_index("c")
        sid = lax.axis_index("s")
        wid = sid * 2 + cid
        base = wid * per_w
        pltpu.sync_copy(idx_hbm.at[wid], idx_v)
        for r in range(8):
            pltpu.async_copy(table_hbm.at[idx_v.at[r]], bufs[r], gsem[r])

        def body(j, _):
            t = j + 4
            for r in range(8):
                @pl.when(jnp.logical_and(lax.rem(t, 8) == r,
                                         jnp.logical_and(t >= 8, t < nch)))
                def _(r=r):
                    pltpu.make_async_copy(
                        bufs[r], out_hbm.at[pl.ds(base + (t - 8) * _GCH, _GCH)],
                        ssem[r]).wait()
                    pltpu.async_copy(table_hbm.at[idx_v.at[t]], bufs[r], gsem[r])

            for r in range(8):
                @pl.when(lax.rem(j, 8) == r)
                def _(r=r):
                    pltpu.make_async_copy(
                        table_hbm.at[idx_v.at[j]], bufs[r], gsem[r]).wait()
                    pltpu.async_copy(
                        bufs[r], out_hbm.at[pl.ds(base + j * _GCH, _GCH)],
                        ssem[r])
            return 0

        lax.fori_loop(0, nch, body, 0)
        for r in range(8):
            pltpu.make_async_copy(
                bufs[r], out_hbm.at[pl.ds(base, _GCH)], ssem[r]).wait()

    return k(table, idx3)


def _mlp_body(g_ref, a_ref, w1, b1, w2, b2, w3, b3, w4, b4, w5, b5,
              m1, n1, m2, n2, m3, n3, o_ref):
    g = g_ref[...]
    geo = g[:, :, 0:3]
    attr = g[:, :, 3:4]
    center = jnp.mean(geo, axis=1, keepdims=True)
    gc = geo - center
    norm = jnp.max(jnp.sqrt(jnp.sum(gc * gc, axis=2, keepdims=True)),
                   axis=1, keepdims=True)
    gn = gc / (norm + jnp.float32(1e-8))
    h = jnp.concatenate([gn, attr], axis=2)
    h = h.reshape(_TM2 * _K, 4)
    for w, bb in ((w1, b1), (w2, b2), (w3, b3), (w4, b4), (w5, b5)):
        h = jnp.maximum(
            jnp.dot(h, w[...], preferred_element_type=jnp.float32) + bb[...], 0.0)
    f = jnp.max(h.reshape(_TM2, _K, 128), axis=1)
    h2 = jnp.maximum(
        jnp.dot(f, m1[...], preferred_element_type=jnp.float32) + n1[...], 0.0)
    h3 = jnp.maximum(
        jnp.dot(h2, m2[...], preferred_element_type=jnp.float32) + n2[...], 0.0)
    ms = jnp.dot(h3, m3[...], preferred_element_type=jnp.float32) + n3[...]
    mu = ms[:, 0:1]
    sig = jnp.clip(jnp.exp(ms[:, 1:2]), jnp.float32(1e-6), jnp.float32(1e10))
    a = a_ref[...]
    inv_sqrt2 = jnp.float32(1.0) / jnp.sqrt(jnp.float32(2.0))

    def cdf(x):
        return 0.5 * (1.0 + lax.erf(x * inv_sqrt2))

    probs = cdf((a + 0.5 - mu) / sig) - cdf((a - 0.5 - mu) / sig)
    bits = jnp.clip(-jnp.log(probs + jnp.float32(1e-10)) / jnp.log(jnp.float32(2.0)),
                    0.0, 50.0)
    o_ref[...] = jnp.sum(bits).reshape(1, 1, 1)


def _mlp_call(grouped, tattr, ws_and_bs):
    nb = grouped.shape[0] // _TM2

    def full(arr):
        nd = arr.ndim
        return pl.BlockSpec(arr.shape, lambda i, _nd=nd: (0,) * _nd)

    return pl.pallas_call(
        _mlp_body,
        grid=(nb,),
        in_specs=[
            pl.BlockSpec((_TM2, _K, 16), lambda i: (i, 0, 0)),
            pl.BlockSpec((_TM2, 1), lambda i: (i, 0)),
        ] + [full(a) for a in ws_and_bs],
        out_specs=pl.BlockSpec((1, 1, 1), lambda i: (i, 0, 0)),
        out_shape=jax.ShapeDtypeStruct((nb, 1, 1), jnp.float32),
    )(grouped, tattr, *ws_and_bs)


def kernel(batch_x, pt_W1, pt_b1, pt_W2, pt_b2, pt_W3, pt_b3, pt_W4, pt_b4,
           pt_W5, pt_b5, ms_W1, ms_b1, ms_W2, ms_b2, ms_W3, ms_b3):
    ctx_t = jnp.transpose(batch_x[:, :, :3], (0, 2, 1))
    idx_parts = []
    for (L, start, Mw) in _WINDOWS:
        targets = batch_x[:, start:start + Mw, :]
        idx_parts.append(_knn_window(targets, ctx_t, L))
    idx = jnp.concatenate(idx_parts, axis=1)

    table = jnp.pad(batch_x.reshape(_B * _N, 4), ((0, 0), (0, 12)))
    nch = (_B * _M * _K) // (_NW * _GCH)
    idx3 = idx.reshape(_NW, nch, _GCH)
    rows = _sc_gather(table, idx3)

    grouped = rows.reshape(_B * _M, _K, 16)
    tattr = batch_x[:, 256:, 3:].reshape(_B * _M, 1)
    wbs = [pt_W1, pt_b1.reshape(1, -1), pt_W2, pt_b2.reshape(1, -1),
           pt_W3, pt_b3.reshape(1, -1), pt_W4, pt_b4.reshape(1, -1),
           pt_W5, pt_b5.reshape(1, -1), ms_W1, ms_b1.reshape(1, -1),
           ms_W2, ms_b2.reshape(1, -1), ms_W3, ms_b3.reshape(1, -1)]
    partial = _mlp_call(grouped, tattr, wbs)
    return jnp.sum(partial)


# speedup vs baseline: 1.0152x; 1.0152x over previous
import functools

import jax
import jax.numpy as jnp
import numpy as np
from jax import lax
from jax.experimental import pallas as pl
from jax.experimental.pallas import tpu as pltpu
from jax.experimental.pallas import tpu_sc as plsc

_K = 16
_N = 8192
_B = 2
_TM = 256
_TM2 = 256
_NW = 32
_GCH = 128


def _window_list():
    out = []
    ws = 256
    cursor = 256
    while cursor < _N:
        ws = min(ws * 2, 1024)
        out.append((cursor, cursor, min(ws, _N - cursor)))
        cursor += ws
    return out


_WINDOWS = _window_list()
_M = sum(w[2] for w in _WINDOWS)


def _knn_body(t_ref, c_ref, o_ref, *, L, TM):
    b = pl.program_id(0)
    t = t_ref[0]
    t3 = t[:, 0:3]
    c3 = c_ref[0]
    tsq = jnp.sum(t3 * t3, axis=1, keepdims=True)
    ta = jnp.concatenate(
        [t3, tsq, jnp.ones((TM, 1), jnp.float32)], axis=1)
    csq = jnp.sum(c3 * c3, axis=0, keepdims=True)
    cb = jnp.concatenate(
        [-2.0 * c3, jnp.ones((1, L), jnp.float32), csq], axis=0)
    d = jnp.maximum(
        jnp.dot(ta, cb, preferred_element_type=jnp.float32), 0.0)
    iota = lax.broadcasted_iota(jnp.int32, (TM, L), 1)
    di = (lax.bitcast_convert_type(d, jnp.int32) & jnp.int32(~8191)) | iota
    L4 = L // 4
    a0 = di[:, 0:L4]
    a1 = di[:, L4:2 * L4]
    a2 = di[:, 2 * L4:3 * L4]
    a3 = di[:, 3 * L4:L]
    lo0 = jnp.minimum(a0, a1)
    hi0 = jnp.maximum(a0, a1)
    lo1 = jnp.minimum(a2, a3)
    hi1 = jnp.maximum(a2, a3)
    e = jnp.minimum(lo0, lo1)
    t1 = jnp.maximum(lo0, lo1)
    t2 = jnp.minimum(hi0, hi1)
    p3 = jnp.maximum(hi0, hi1)
    p1 = jnp.minimum(t1, t2)
    p2 = jnp.maximum(t1, t2)
    imax = jnp.int32(0x7FFFFFFF)
    cols = []
    for _ in range(_K):
        m = jnp.min(e, axis=1, keepdims=True)
        mask = e == m
        e = jnp.where(mask, p1, e)
        p1 = jnp.where(mask, p2, p1)
        p2 = jnp.where(mask, p3, p2)
        p3 = jnp.where(mask, imax, p3)
        cols.append(m & jnp.int32(8191))
    idx = jnp.concatenate(cols, axis=1)
    o_ref[0] = idx + b * _N


def _knn_window(targets, ctx_t, L):
    Mw = targets.shape[1]
    TM = min(_TM, Mw)
    grid = (_B, Mw // TM)
    return pl.pallas_call(
        functools.partial(_knn_body, L=L, TM=TM),
        grid=grid,
        in_specs=[
            pl.BlockSpec((1, TM, 4), lambda b, i: (b, i, 0)),
            pl.BlockSpec((1, 3, L), lambda b, i: (b, 0, 0)),
        ],
        out_specs=pl.BlockSpec((1, TM, _K), lambda b, i: (b, i, 0)),
        out_shape=jax.ShapeDtypeStruct((_B, Mw, _K), jnp.int32),
    )(targets, ctx_t)


def _sc_gather(table, idx3):
    nch = idx3.shape[1]
    total = _NW * nch * _GCH
    per_w = nch * _GCH
    mesh = plsc.VectorSubcoreMesh(core_axis_name="c", subcore_axis_name="s")

    @functools.partial(
        pl.kernel,
        mesh=mesh,
        compiler_params=pltpu.CompilerParams(use_tc_tiling_on_sc=False),
        out_type=jax.ShapeDtypeStruct((total, 16), jnp.float32),
        scratch_types=[
            pltpu.VMEM((nch, _GCH), jnp.int32),
        ] + [pltpu.VMEM((_GCH, 16), jnp.float32)] * 8
          + [pltpu.SemaphoreType.DMA] * 16,
    )
    def k(table_hbm, idx_hbm, out_hbm, idx_v, *bufs_and_sems):
        bufs = bufs_and_sems[0:8]
        gsem = bufs_and_sems[8:16]
        ssem = bufs_and_sems[16:24]
        cid = lax.axis_index("c")
        sid = lax.axis_index("s")
        wid = sid * 2 + cid
        base = wid * per_w
        pltpu.sync_copy(idx_hbm.at[wid], idx_v)
        for r in range(8):
            pltpu.async_copy(table_hbm.at[idx_v.at[r]], bufs[r], gsem[r])

        def body(j, _):
            t = j + 4
            for r in range(8):
                @pl.when(jnp.logical_and(lax.rem(t, 8) == r,
                                         jnp.logical_and(t >= 8, t < nch)))
                def _(r=r):
                    pltpu.make_async_copy(
                        bufs[r], out_hbm.at[pl.ds(base + (t - 8) * _GCH, _GCH)],
                        ssem[r]).wait()
                    pltpu.async_copy(table_hbm.at[idx_v.at[t]], bufs[r], gsem[r])

            for r in range(8):
                @pl.when(lax.rem(j, 8) == r)
                def _(r=r):
                    pltpu.make_async_copy(
                        table_hbm.at[idx_v.at[j]], bufs[r], gsem[r]).wait()
                    pltpu.async_copy(
                        bufs[r], out_hbm.at[pl.ds(base + j * _GCH, _GCH)],
                        ssem[r])
            return 0

        lax.fori_loop(0, nch, body, 0)
        for r in range(8):
            pltpu.make_async_copy(
                bufs[r], out_hbm.at[pl.ds(base, _GCH)], ssem[r]).wait()

    return k(table, idx3)


def _tree_fold(x, op):
    k = x.shape[1]
    while k > 1:
        k //= 2
        x = op(x[:, :k, :], x[:, k:2 * k, :])
    return x


def _mlp_body(g_ref, a_ref, w1, b1, w2, b2, w3, b3, w4, b4, w5, b5,
              m1, n1, m2, n2, m3, n3, o_ref):
    g = g_ref[...]
    geo = g[:, :, 0:3]
    attr = g[:, :, 3:4]
    center = _tree_fold(geo, jnp.add) * jnp.float32(1.0 / _K)
    gc = geo - center
    nsq = jnp.sum(gc * gc, axis=2, keepdims=True)
    norm = jnp.sqrt(_tree_fold(nsq, jnp.maximum))
    gn = gc / (norm + jnp.float32(1e-8))
    h = jnp.concatenate([gn, attr], axis=2)
    h = h.reshape(_TM2 * _K, 4)
    for w, bb in ((w1, b1), (w2, b2), (w3, b3), (w4, b4), (w5, b5)):
        h = jnp.maximum(
            jnp.dot(h, w[...], preferred_element_type=jnp.float32) + bb[...], 0.0)
    f = _tree_fold(h.reshape(_TM2, _K, 128), jnp.maximum)[:, 0, :]
    h2 = jnp.maximum(
        jnp.dot(f, m1[...], preferred_element_type=jnp.float32) + n1[...], 0.0)
    h3 = jnp.maximum(
        jnp.dot(h2, m2[...], preferred_element_type=jnp.float32) + n2[...], 0.0)
    ms = jnp.dot(h3, m3[...], preferred_element_type=jnp.float32) + n3[...]
    mu = ms[:, 0:1]
    sig = jnp.clip(jnp.exp(ms[:, 1:2]), jnp.float32(1e-6), jnp.float32(1e10))
    a = a_ref[...]
    inv_sqrt2 = jnp.float32(1.0) / jnp.sqrt(jnp.float32(2.0))

    def cdf(x):
        return 0.5 * (1.0 + lax.erf(x * inv_sqrt2))

    probs = cdf((a + 0.5 - mu) / sig) - cdf((a - 0.5 - mu) / sig)
    bits = jnp.clip(-jnp.log(probs + jnp.float32(1e-10)) / jnp.log(jnp.float32(2.0)),
                    0.0, 50.0)
    o_ref[...] = jnp.sum(bits).reshape(1, 1, 1)


def _mlp_call(grouped, tattr, ws_and_bs):
    nb = grouped.shape[0] // _TM2

    def full(arr):
        nd = arr.ndim
        return pl.BlockSpec(arr.shape, lambda i, _nd=nd: (0,) * _nd)

    return pl.pallas_call(
        _mlp_body,
        grid=(nb,),
        in_specs=[
            pl.BlockSpec((_TM2, _K, 16), lambda i: (i, 0, 0)),
            pl.BlockSpec((_TM2, 1), lambda i: (i, 0)),
        ] + [full(a) for a in ws_and_bs],
        out_specs=pl.BlockSpec((1, 1, 1), lambda i: (i, 0, 0)),
        out_shape=jax.ShapeDtypeStruct((nb, 1, 1), jnp.float32),
    )(grouped, tattr, *ws_and_bs)


def kernel(batch_x, pt_W1, pt_b1, pt_W2, pt_b2, pt_W3, pt_b3, pt_W4, pt_b4,
           pt_W5, pt_b5, ms_W1, ms_b1, ms_W2, ms_b2, ms_W3, ms_b3):
    ctx_t = jnp.transpose(batch_x[:, :, :3], (0, 2, 1))
    idx_parts = []
    for (L, start, Mw) in _WINDOWS:
        targets = batch_x[:, start:start + Mw, :]
        idx_parts.append(_knn_window(targets, ctx_t, L))
    idx = jnp.concatenate(idx_parts, axis=1)

    table = jnp.pad(batch_x.reshape(_B * _N, 4), ((0, 0), (0, 12)))
    nch = (_B * _M * _K) // (_NW * _GCH)
    idx3 = idx.reshape(_NW, nch, _GCH)
    rows = _sc_gather(table, idx3)

    grouped = rows.reshape(_B * _M, _K, 16)
    tattr = batch_x[:, 256:, 3:].reshape(_B * _M, 1)
    wbs = [pt_W1, pt_b1.reshape(1, -1), pt_W2, pt_b2.reshape(1, -1),
           pt_W3, pt_b3.reshape(1, -1), pt_W4, pt_b4.reshape(1, -1),
           pt_W5, pt_b5.reshape(1, -1), ms_W1, ms_b1.reshape(1, -1),
           ms_W2, ms_b2.reshape(1, -1), ms_W3, ms_b3.reshape(1, -1)]
    partial = _mlp_call(grouped, tattr, wbs)
    return jnp.sum(partial)


# speedup vs baseline: 1.0443x; 1.0286x over previous
import functools

import jax
import jax.numpy as jnp
import numpy as np
from jax import lax
from jax.experimental import pallas as pl
from jax.experimental.pallas import tpu as pltpu
from jax.experimental.pallas import tpu_sc as plsc

_K = 16
_N = 8192
_B = 2
_TM = 256
_TM2 = 512
_NW = 32
_GCH = 128


def _window_list():
    out = []
    ws = 256
    cursor = 256
    while cursor < _N:
        ws = min(ws * 2, 1024)
        out.append((cursor, cursor, min(ws, _N - cursor)))
        cursor += ws
    return out


_WINDOWS = _window_list()
_M = sum(w[2] for w in _WINDOWS)


def _knn_body(t_ref, c_ref, o_ref, *, L, TM):
    b = pl.program_id(0)
    t = t_ref[0]
    t3 = t[:, 0:3]
    c3 = c_ref[0]
    tsq = jnp.sum(t3 * t3, axis=1, keepdims=True)
    ta = jnp.concatenate(
        [t3, tsq, jnp.ones((TM, 1), jnp.float32)], axis=1)
    csq = jnp.sum(c3 * c3, axis=0, keepdims=True)
    cb = jnp.concatenate(
        [-2.0 * c3, jnp.ones((1, L), jnp.float32), csq], axis=0)
    d = jnp.maximum(
        jnp.dot(ta, cb, preferred_element_type=jnp.float32), 0.0)
    iota = lax.broadcasted_iota(jnp.int32, (TM, L), 1)
    di = (lax.bitcast_convert_type(d, jnp.int32) & jnp.int32(~8191)) | iota
    L4 = L // 4
    a0 = di[:, 0:L4]
    a1 = di[:, L4:2 * L4]
    a2 = di[:, 2 * L4:3 * L4]
    a3 = di[:, 3 * L4:L]
    lo0 = jnp.minimum(a0, a1)
    hi0 = jnp.maximum(a0, a1)
    lo1 = jnp.minimum(a2, a3)
    hi1 = jnp.maximum(a2, a3)
    e = jnp.minimum(lo0, lo1)
    t1 = jnp.maximum(lo0, lo1)
    t2 = jnp.minimum(hi0, hi1)
    p3 = jnp.maximum(hi0, hi1)
    p1 = jnp.minimum(t1, t2)
    p2 = jnp.maximum(t1, t2)
    imax = jnp.int32(0x7FFFFFFF)
    cols = []
    for _ in range(_K):
        m = jnp.min(e, axis=1, keepdims=True)
        mask = e == m
        e = jnp.where(mask, p1, e)
        p1 = jnp.where(mask, p2, p1)
        p2 = jnp.where(mask, p3, p2)
        p3 = jnp.where(mask, imax, p3)
        cols.append(m & jnp.int32(8191))
    idx = jnp.concatenate(cols, axis=1)
    o_ref[0] = idx + b * _N


def _knn_window(targets, ctx_t, L):
    Mw = targets.shape[1]
    TM = min(_TM, Mw)
    grid = (_B, Mw // TM)
    return pl.pallas_call(
        functools.partial(_knn_body, L=L, TM=TM),
        grid=grid,
        in_specs=[
            pl.BlockSpec((1, TM, 4), lambda b, i: (b, i, 0)),
            pl.BlockSpec((1, 3, L), lambda b, i: (b, 0, 0)),
        ],
        out_specs=pl.BlockSpec((1, TM, _K), lambda b, i: (b, i, 0)),
        out_shape=jax.ShapeDtypeStruct((_B, Mw, _K), jnp.int32),
    )(targets, ctx_t)


def _sc_gather(table, idx3):
    nch = idx3.shape[1]
    total = _NW * nch * _GCH
    per_w = nch * _GCH
    mesh = plsc.VectorSubcoreMesh(core_axis_name="c", subcore_axis_name="s")

    @functools.partial(
        pl.kernel,
        mesh=mesh,
        compiler_params=pltpu.CompilerParams(use_tc_tiling_on_sc=False),
        out_type=jax.ShapeDtypeStruct((total, 16), jnp.float32),
        scratch_types=[
            pltpu.VMEM((nch, _GCH), jnp.int32),
        ] + [pltpu.VMEM((_GCH, 16), jnp.float32)] * 8
          + [pltpu.SemaphoreType.DMA] * 16,
    )
    def k(table_hbm, idx_hbm, out_hbm, idx_v, *bufs_and_sems):
        bufs = bufs_and_sems[0:8]
        gsem = bufs_and_sems[8:16]
        ssem = bufs_and_sems[16:24]
        cid = lax.axis_index("c")
        sid = lax.axis_index("s")
        wid = sid * 2 + cid
        base = wid * per_w
        pltpu.sync_copy(idx_hbm.at[wid], idx_v)
        for r in range(8):
            pltpu.async_copy(table_hbm.at[idx_v.at[r]], bufs[r], gsem[r])

        def body(j, _):
            t = j + 4
            for r in range(8):
                @pl.when(jnp.logical_and(lax.rem(t, 8) == r,
                                         jnp.logical_and(t >= 8, t < nch)))
                def _(r=r):
                    pltpu.make_async_copy(
                        bufs[r], out_hbm.at[pl.ds(base + (t - 8) * _GCH, _GCH)],
                        ssem[r]).wait()
                    pltpu.async_copy(table_hbm.at[idx_v.at[t]], bufs[r], gsem[r])

            for r in range(8):
                @pl.when(lax.rem(j, 8) == r)
                def _(r=r):
                    pltpu.make_async_copy(
                        table_hbm.at[idx_v.at[j]], bufs[r], gsem[r]).wait()
                    pltpu.async_copy(
                        bufs[r], out_hbm.at[pl.ds(base + j * _GCH, _GCH)],
                        ssem[r])
            return 0

        lax.fori_loop(0, nch, body, 0)
        for r in range(8):
            pltpu.make_async_copy(
                bufs[r], out_hbm.at[pl.ds(base, _GCH)], ssem[r]).wait()

    return k(table, idx3)


def _tree_fold(x, op):
    k = x.shape[1]
    while k > 1:
        k //= 2
        x = op(x[:, :k, :], x[:, k:2 * k, :])
    return x


def _mlp_body(g_ref, a_ref, w1, b1, w2, b2, w3, b3, w4, b4, w5, b5,
              m1, n1, m2, n2, m3, n3, o_ref):
    g = g_ref[...]
    geo = g[:, :, 0:3]
    attr = g[:, :, 3:4]
    center = _tree_fold(geo, jnp.add) * jnp.float32(1.0 / _K)
    gc = geo - center
    nsq = jnp.sum(gc * gc, axis=2, keepdims=True)
    norm = jnp.sqrt(_tree_fold(nsq, jnp.maximum))
    gn = gc / (norm + jnp.float32(1e-8))
    h = jnp.concatenate([gn, attr], axis=2)
    h = h.reshape(_TM2 * _K, 4)
    for w, bb in ((w1, b1), (w2, b2), (w3, b3), (w4, b4), (w5, b5)):
        h = jnp.maximum(
            jnp.dot(h, w[...], preferred_element_type=jnp.float32) + bb[...], 0.0)
    f = _tree_fold(h.reshape(_TM2, _K, 128), jnp.maximum)[:, 0, :]
    h2 = jnp.maximum(
        jnp.dot(f, m1[...], preferred_element_type=jnp.float32) + n1[...], 0.0)
    h3 = jnp.maximum(
        jnp.dot(h2, m2[...], preferred_element_type=jnp.float32) + n2[...], 0.0)
    ms = jnp.dot(h3, m3[...], preferred_element_type=jnp.float32) + n3[...]
    mu = ms[:, 0:1]
    sig = jnp.clip(jnp.exp(ms[:, 1:2]), jnp.float32(1e-6), jnp.float32(1e10))
    a = a_ref[...]
    inv_sqrt2 = jnp.float32(1.0) / jnp.sqrt(jnp.float32(2.0))

    def cdf(x):
        return 0.5 * (1.0 + lax.erf(x * inv_sqrt2))

    probs = cdf((a + 0.5 - mu) / sig) - cdf((a - 0.5 - mu) / sig)
    bits = jnp.clip(-jnp.log(probs + jnp.float32(1e-10)) / jnp.log(jnp.float32(2.0)),
                    0.0, 50.0)
    o_ref[...] = jnp.sum(bits).reshape(1, 1, 1)


def _mlp_call(grouped, tattr, ws_and_bs):
    nb = grouped.shape[0] // _TM2

    def full(arr):
        nd = arr.ndim
        return pl.BlockSpec(arr.shape, lambda i, _nd=nd: (0,) * _nd)

    return pl.pallas_call(
        _mlp_body,
        grid=(nb,),
        in_specs=[
            pl.BlockSpec((_TM2, _K, 16), lambda i: (i, 0, 0)),
            pl.BlockSpec((_TM2, 1), lambda i: (i, 0)),
        ] + [full(a) for a in ws_and_bs],
        out_specs=pl.BlockSpec((1, 1, 1), lambda i: (i, 0, 0)),
        out_shape=jax.ShapeDtypeStruct((nb, 1, 1), jnp.float32),
    )(grouped, tattr, *ws_and_bs)


def kernel(batch_x, pt_W1, pt_b1, pt_W2, pt_b2, pt_W3, pt_b3, pt_W4, pt_b4,
           pt_W5, pt_b5, ms_W1, ms_b1, ms_W2, ms_b2, ms_W3, ms_b3):
    ctx_t = jnp.transpose(batch_x[:, :, :3], (0, 2, 1))
    idx_parts = []
    for (L, start, Mw) in _WINDOWS:
        targets = batch_x[:, start:start + Mw, :]
        idx_parts.append(_knn_window(targets, ctx_t, L))
    idx = jnp.concatenate(idx_parts, axis=1)

    table = jnp.pad(batch_x.reshape(_B * _N, 4), ((0, 0), (0, 12)))
    nch = (_B * _M * _K) // (_NW * _GCH)
    idx3 = idx.reshape(_NW, nch, _GCH)
    rows = _sc_gather(table, idx3)

    grouped = rows.reshape(_B * _M, _K, 16)
    tattr = batch_x[:, 256:, 3:].reshape(_B * _M, 1)
    wbs = [pt_W1, pt_b1.reshape(1, -1), pt_W2, pt_b2.reshape(1, -1),
           pt_W3, pt_b3.reshape(1, -1), pt_W4, pt_b4.reshape(1, -1),
           pt_W5, pt_b5.reshape(1, -1), ms_W1, ms_b1.reshape(1, -1),
           ms_W2, ms_b2.reshape(1, -1), ms_W3, ms_b3.reshape(1, -1)]
    partial = _mlp_call(grouped, tattr, wbs)
    return jnp.sum(partial)


# speedup vs baseline: 1.0565x; 1.0117x over previous
import functools

import jax
import jax.numpy as jnp
import numpy as np
from jax import lax
from jax.experimental import pallas as pl
from jax.experimental.pallas import tpu as pltpu
from jax.experimental.pallas import tpu_sc as plsc

_K = 16
_N = 8192
_B = 2
_TM = 256
_TM2 = 1984
_NW = 32
_GCH = 128


def _window_list():
    out = []
    ws = 256
    cursor = 256
    while cursor < _N:
        ws = min(ws * 2, 1024)
        out.append((cursor, cursor, min(ws, _N - cursor)))
        cursor += ws
    return out


_WINDOWS = _window_list()
_M = sum(w[2] for w in _WINDOWS)


def _knn_body(t_ref, c_ref, o_ref, *, L, TM):
    b = pl.program_id(0)
    t = t_ref[0]
    t3 = t[:, 0:3]
    c3 = c_ref[0]
    tsq = jnp.sum(t3 * t3, axis=1, keepdims=True)
    ta = jnp.concatenate(
        [t3, tsq, jnp.ones((TM, 1), jnp.float32)], axis=1)
    csq = jnp.sum(c3 * c3, axis=0, keepdims=True)
    cb = jnp.concatenate(
        [-2.0 * c3, jnp.ones((1, L), jnp.float32), csq], axis=0)
    d = jnp.maximum(
        jnp.dot(ta, cb, preferred_element_type=jnp.float32), 0.0)
    iota = lax.broadcasted_iota(jnp.int32, (TM, L), 1)
    di = (lax.bitcast_convert_type(d, jnp.int32) & jnp.int32(~8191)) | iota
    L4 = L // 4
    a0 = di[:, 0:L4]
    a1 = di[:, L4:2 * L4]
    a2 = di[:, 2 * L4:3 * L4]
    a3 = di[:, 3 * L4:L]
    lo0 = jnp.minimum(a0, a1)
    hi0 = jnp.maximum(a0, a1)
    lo1 = jnp.minimum(a2, a3)
    hi1 = jnp.maximum(a2, a3)
    e = jnp.minimum(lo0, lo1)
    t1 = jnp.maximum(lo0, lo1)
    t2 = jnp.minimum(hi0, hi1)
    p3 = jnp.maximum(hi0, hi1)
    p1 = jnp.minimum(t1, t2)
    p2 = jnp.maximum(t1, t2)
    imax = jnp.int32(0x7FFFFFFF)
    cols = []
    for _ in range(_K):
        m = jnp.min(e, axis=1, keepdims=True)
        mask = e == m
        e = jnp.where(mask, p1, e)
        p1 = jnp.where(mask, p2, p1)
        p2 = jnp.where(mask, p3, p2)
        p3 = jnp.where(mask, imax, p3)
        cols.append(m & jnp.int32(8191))
    idx = jnp.concatenate(cols, axis=1)
    o_ref[0] = idx + b * _N


def _knn_window(targets, ctx_t, L):
    Mw = targets.shape[1]
    TM = min(_TM, Mw)
    grid = (_B, Mw // TM)
    return pl.pallas_call(
        functools.partial(_knn_body, L=L, TM=TM),
        grid=grid,
        in_specs=[
            pl.BlockSpec((1, TM, 4), lambda b, i: (b, i, 0)),
            pl.BlockSpec((1, 3, L), lambda b, i: (b, 0, 0)),
        ],
        out_specs=pl.BlockSpec((1, TM, _K), lambda b, i: (b, i, 0)),
        out_shape=jax.ShapeDtypeStruct((_B, Mw, _K), jnp.int32),
    )(targets, ctx_t)


def _sc_gather(table, idx3):
    nch = idx3.shape[1]
    total = _NW * nch * _GCH
    per_w = nch * _GCH
    mesh = plsc.VectorSubcoreMesh(core_axis_name="c", subcore_axis_name="s")

    @functools.partial(
        pl.kernel,
        mesh=mesh,
        compiler_params=pltpu.CompilerParams(use_tc_tiling_on_sc=False),
        out_type=jax.ShapeDtypeStruct((total, 16), jnp.float32),
        scratch_types=[
            pltpu.VMEM((nch, _GCH), jnp.int32),
        ] + [pltpu.VMEM((_GCH, 16), jnp.float32)] * 8
          + [pltpu.SemaphoreType.DMA] * 16,
    )
    def k(table_hbm, idx_hbm, out_hbm, idx_v, *bufs_and_sems):
        bufs = bufs_and_sems[0:8]
        gsem = bufs_and_sems[8:16]
        ssem = bufs_and_sems[16:24]
        cid = lax.axis_index("c")
        sid = lax.axis_index("s")
        wid = sid * 2 + cid
        base = wid * per_w
        pltpu.sync_copy(idx_hbm.at[wid], idx_v)
        for r in range(8):
            pltpu.async_copy(table_hbm.at[idx_v.at[r]], bufs[r], gsem[r])

        def body(j, _):
            t = j + 4
            for r in range(8):
                @pl.when(jnp.logical_and(lax.rem(t, 8) == r,
                                         jnp.logical_and(t >= 8, t < nch)))
                def _(r=r):
                    pltpu.make_async_copy(
                        bufs[r], out_hbm.at[pl.ds(base + (t - 8) * _GCH, _GCH)],
                        ssem[r]).wait()
                    pltpu.async_copy(table_hbm.at[idx_v.at[t]], bufs[r], gsem[r])

            for r in range(8):
                @pl.when(lax.rem(j, 8) == r)
                def _(r=r):
                    pltpu.make_async_copy(
                        table_hbm.at[idx_v.at[j]], bufs[r], gsem[r]).wait()
                    pltpu.async_copy(
                        bufs[r], out_hbm.at[pl.ds(base + j * _GCH, _GCH)],
                        ssem[r])
            return 0

        lax.fori_loop(0, nch, body, 0)
        for r in range(8):
            pltpu.make_async_copy(
                bufs[r], out_hbm.at[pl.ds(base, _GCH)], ssem[r]).wait()

    return k(table, idx3)


def _tree_fold(x, op):
    k = x.shape[1]
    while k > 1:
        k //= 2
        x = op(x[:, :k, :], x[:, k:2 * k, :])
    return x


def _mlp_body(g_ref, a_ref, w1, b1, w2, b2, w3, b3, w4, b4, w5, b5,
              m1, n1, m2, n2, m3, n3, o_ref):
    g = g_ref[...]
    geo = g[:, :, 0:3]
    attr = g[:, :, 3:4]
    center = _tree_fold(geo, jnp.add) * jnp.float32(1.0 / _K)
    gc = geo - center
    nsq = jnp.sum(gc * gc, axis=2, keepdims=True)
    norm = jnp.sqrt(_tree_fold(nsq, jnp.maximum))
    gn = gc / (norm + jnp.float32(1e-8))
    h = jnp.concatenate([gn, attr], axis=2)
    h = h.reshape(_TM2 * _K, 4)
    for w, bb in ((w1, b1), (w2, b2), (w3, b3), (w4, b4), (w5, b5)):
        h = jnp.maximum(
            jnp.dot(h, w[...], preferred_element_type=jnp.float32) + bb[...], 0.0)
    f = _tree_fold(h.reshape(_TM2, _K, 128), jnp.maximum)[:, 0, :]
    h2 = jnp.maximum(
        jnp.dot(f, m1[...], preferred_element_type=jnp.float32) + n1[...], 0.0)
    h3 = jnp.maximum(
        jnp.dot(h2, m2[...], preferred_element_type=jnp.float32) + n2[...], 0.0)
    ms = jnp.dot(h3, m3[...], preferred_element_type=jnp.float32) + n3[...]
    mu = ms[:, 0:1]
    sig = jnp.clip(jnp.exp(ms[:, 1:2]), jnp.float32(1e-6), jnp.float32(1e10))
    a = a_ref[...]
    inv_sqrt2 = jnp.float32(1.0) / jnp.sqrt(jnp.float32(2.0))

    def cdf(x):
        return 0.5 * (1.0 + lax.erf(x * inv_sqrt2))

    probs = cdf((a + 0.5 - mu) / sig) - cdf((a - 0.5 - mu) / sig)
    bits = jnp.clip(-jnp.log(probs + jnp.float32(1e-10)) / jnp.log(jnp.float32(2.0)),
                    0.0, 50.0)
    o_ref[...] = jnp.sum(bits).reshape(1, 1, 1)


def _mlp_call(grouped, tattr, ws_and_bs):
    nb = grouped.shape[0] // _TM2

    def full(arr):
        nd = arr.ndim
        return pl.BlockSpec(arr.shape, lambda i, _nd=nd: (0,) * _nd)

    return pl.pallas_call(
        _mlp_body,
        grid=(nb,),
        in_specs=[
            pl.BlockSpec((_TM2, _K, 16), lambda i: (i, 0, 0)),
            pl.BlockSpec((_TM2, 1), lambda i: (i, 0)),
        ] + [full(a) for a in ws_and_bs],
        out_specs=pl.BlockSpec((1, 1, 1), lambda i: (i, 0, 0)),
        out_shape=jax.ShapeDtypeStruct((nb, 1, 1), jnp.float32),
    )(grouped, tattr, *ws_and_bs)


def kernel(batch_x, pt_W1, pt_b1, pt_W2, pt_b2, pt_W3, pt_b3, pt_W4, pt_b4,
           pt_W5, pt_b5, ms_W1, ms_b1, ms_W2, ms_b2, ms_W3, ms_b3):
    ctx_t = jnp.transpose(batch_x[:, :, :3], (0, 2, 1))
    idx_parts = []
    for (L, start, Mw) in _WINDOWS:
        targets = batch_x[:, start:start + Mw, :]
        idx_parts.append(_knn_window(targets, ctx_t, L))
    idx = jnp.concatenate(idx_parts, axis=1)

    table = jnp.pad(batch_x.reshape(_B * _N, 4), ((0, 0), (0, 12)))
    nch = (_B * _M * _K) // (_NW * _GCH)
    idx3 = idx.reshape(_NW, nch, _GCH)
    rows = _sc_gather(table, idx3)

    grouped = rows.reshape(_B * _M, _K, 16)
    tattr = batch_x[:, 256:, 3:].reshape(_B * _M, 1)
    wbs = [pt_W1, pt_b1.reshape(1, -1), pt_W2, pt_b2.reshape(1, -1),
           pt_W3, pt_b3.reshape(1, -1), pt_W4, pt_b4.reshape(1, -1),
           pt_W5, pt_b5.reshape(1, -1), ms_W1, ms_b1.reshape(1, -1),
           ms_W2, ms_b2.reshape(1, -1), ms_W3, ms_b3.reshape(1, -1)]
    partial = _mlp_call(grouped, tattr, wbs)
    return jnp.sum(partial)
